# Initial kernel scaffold; baseline (speedup 1.0000x reference)
import jax
import jax.numpy as jnp
from jax.experimental import pallas as pl


def kernel(h, e, edge_index, n1_w0, n1_b0, n1_w1, n1_b1, conv1_bias, p1_w, p1_b, n2_w0, n2_b0, n2_w1, n2_b1, conv2_bias, p2_w, p2_b, fc1_w, fc1_b, fc2_w, fc2_b, fc3_w, fc3_b):
    raise NotImplementedError("write your pallas kernel here")


# speedup vs baseline: 8.4153x; 8.4153x over previous
import dataclasses
import functools
import jax
import jax.numpy as jnp
from jax import lax
from jax.experimental import pallas as pl
from jax.experimental.pallas import tpu as pltpu
from jax.experimental.pallas import tpu_sc as plsc

N = 10000
NP = 10240
E = 160000
EP = 163840
ER = EP // 128
WPW = ER // 32
ZR = NP // 16
F32 = jnp.float32

_mesh = plsc.VectorSubcoreMesh(core_axis_name="c", subcore_axis_name="s")

_sc_params = pltpu.CompilerParams()
for _f, _v in (("needs_layout_passes", False),
               ("use_tc_tiling_on_sc", False)):
    if _f in pltpu.CompilerParams.__dataclass_fields__:
        _sc_params = dataclasses.replace(_sc_params, **{_f: _v})


def _acc_sds():
    return jax.ShapeDtypeStruct((2 * NP, 16), F32)


@functools.partial(
    pl.kernel,
    out_type=(_acc_sds(), _acc_sds(), _acc_sds()),
    mesh=_mesh,
    compiler_params=_sc_params,
    scratch_types=[
        pltpu.VMEM_SHARED((NP, 16), F32),
        pltpu.VMEM_SHARED((NP, 16), F32),
        pltpu.VMEM_SHARED((NP, 16), F32),
        pltpu.VMEM((WPW, 128), jnp.int32),
        pltpu.VMEM((WPW, 128), jnp.int32),
        pltpu.VMEM((WPW, 128), F32),
        pltpu.VMEM((128, 16), F32),
        pltpu.VMEM((128, 16), F32),
        pltpu.VMEM((128, 16), F32),
    ],
)
def _sc_conv(uv_hbm, ind_hbm, a_hbm, src_hbm, dst_hbm, z_hbm,
             accm_hbm, acco_hbm, acci_hbm,
             shm, sho, shi, idx_s, idx_d, ab, rm, ra, rb):
    c = lax.axis_index("c")
    s = lax.axis_index("s")
    wid = s * 2 + c
    pltpu.sync_copy(z_hbm.at[pl.ds(s * ZR, ZR)], shm.at[pl.ds(s * ZR, ZR)])
    pltpu.sync_copy(z_hbm.at[pl.ds(s * ZR, ZR)], sho.at[pl.ds(s * ZR, ZR)])
    pltpu.sync_copy(z_hbm.at[pl.ds(s * ZR, ZR)], shi.at[pl.ds(s * ZR, ZR)])
    plsc.subcore_barrier()

    base = wid * WPW
    pltpu.sync_copy(src_hbm.at[pl.ds(base, WPW)], idx_s)
    pltpu.sync_copy(dst_hbm.at[pl.ds(base, WPW)], idx_d)
    pltpu.sync_copy(a_hbm.at[pl.ds(base, WPW)], ab)

    i16 = lax.broadcasted_iota(jnp.int32, (16,), 0)
    maskf = (i16 < 8).astype(F32)
    invf = 1.0 - maskf

    @pl.loop(0, WPW)
    def _(j):
        pltpu.sync_copy(uv_hbm.at[idx_s.at[j]], rm)

        @pl.loop(0, 8)
        def _(g):
            av = ab[j, pl.ds(g * 16, 16)]
            for k in range(16):
                i = g * 16 + k
                rm[i, :] = rm[i, :] * (maskf * av[k] + invf)

        pltpu.sync_copy(rm, shm.at[idx_d.at[j]], add=True)
        pltpu.sync_copy(ind_hbm.at[idx_s.at[j]], ra)
        pltpu.sync_copy(ra, shi.at[idx_d.at[j]], add=True)
        pltpu.sync_copy(ind_hbm.at[idx_d.at[j]], rb)
        pltpu.sync_copy(rb, sho.at[idx_s.at[j]], add=True)

    plsc.subcore_barrier()
    off = c * NP + s * ZR
    pltpu.sync_copy(shm.at[pl.ds(s * ZR, ZR)], accm_hbm.at[pl.ds(off, ZR)])
    pltpu.sync_copy(sho.at[pl.ds(s * ZR, ZR)], acco_hbm.at[pl.ds(off, ZR)])
    pltpu.sync_copy(shi.at[pl.ds(s * ZR, ZR)], acci_hbm.at[pl.ds(off, ZR)])


@functools.partial(
    pl.kernel,
    out_type=_acc_sds(),
    mesh=_mesh,
    compiler_params=_sc_params,
    scratch_types=[
        pltpu.VMEM_SHARED((NP, 16), F32),
        pltpu.VMEM((WPW, 128), jnp.int32),
        pltpu.VMEM((WPW, 128), jnp.int32),
        pltpu.VMEM((128, 16), F32),
    ],
)
def _sc_agg(q_hbm, src_hbm, dst_hbm, z_hbm, acca_hbm, sha, idx_s, idx_d, rq):
    c = lax.axis_index("c")
    s = lax.axis_index("s")
    wid = s * 2 + c
    pltpu.sync_copy(z_hbm.at[pl.ds(s * ZR, ZR)], sha.at[pl.ds(s * ZR, ZR)])
    plsc.subcore_barrier()

    base = wid * WPW
    pltpu.sync_copy(src_hbm.at[pl.ds(base, WPW)], idx_s)
    pltpu.sync_copy(dst_hbm.at[pl.ds(base, WPW)], idx_d)

    @pl.loop(0, WPW)
    def _(j):
        pltpu.sync_copy(q_hbm.at[idx_s.at[j]], rq)
        pltpu.sync_copy(rq, sha.at[idx_d.at[j]], add=True)

    plsc.subcore_barrier()
    off = c * NP + s * ZR
    pltpu.sync_copy(sha.at[pl.ds(s * ZR, ZR)], acca_hbm.at[pl.ds(off, ZR)])


def _tca_body(h_ref, w_ref, e_ref, w0_ref, b0_ref, uv_ref, a_ref):
    uv_ref[...] = jnp.dot(h_ref[...], w_ref[...], preferred_element_type=F32)
    a_ref[...] = jnp.maximum(e_ref[...] * w0_ref[0, 0] + b0_ref[0, 0], 0.0)


_tc_a = pl.pallas_call(
    _tca_body,
    out_shape=(jax.ShapeDtypeStruct((NP, 16), F32),
               jax.ShapeDtypeStruct((ER, 128), F32)),
)


def _tcb_body(accm_ref, acco_ref, acci_ref, bias_ref, pw_ref,
              h1_ref, qtab_ref, rdin_ref):
    m = accm_ref[:NP] + accm_ref[NP:]
    h1 = m[:, :8] + m[:, 8:] + bias_ref[...]
    h1_ref[...] = h1
    dego = jnp.maximum(acco_ref[:NP, 0:1] + acco_ref[NP:, 0:1], 1.0)
    degi = jnp.maximum(acci_ref[:NP, 0:1] + acci_ref[NP:, 0:1], 1.0)
    rdin_ref[...] = jnp.broadcast_to(lax.rsqrt(degi), (NP, 16))
    q = jnp.sum(h1 * pw_ref[...], axis=1, keepdims=True)
    qtab_ref[...] = jnp.broadcast_to(q * lax.rsqrt(dego), (NP, 16))


_tc_b = pl.pallas_call(
    _tcb_body,
    out_shape=(jax.ShapeDtypeStruct((NP, 8), F32),
               jax.ShapeDtypeStruct((NP, 16), F32),
               jax.ShapeDtypeStruct((NP, 16), F32)),
)


def _tcc_body(acca_ref, rdin_ref, b_ref, score_ref):
    agg = acca_ref[:NP] + acca_ref[NP:]
    sv = agg * rdin_ref[...] + b_ref[0, 0]
    score_ref[...] = 1.0 / (1.0 + jnp.exp(-sv))


_tc_c = pl.pallas_call(
    _tcc_body,
    out_shape=jax.ShapeDtypeStruct((NP, 16), F32),
)


def _tcd_body(h1_ref, score_ref, t_ref, ib_ref, w_ref, e_ref, w0_ref, b0_ref,
              h1p_ref, uv2_ref, nmtab_ref, a2_ref, nm_ref):
    sc = score_ref[:, 0:1]
    iota = lax.broadcasted_iota(jnp.int32, (NP, 1), 0)
    t = t_ref[0, 0]
    ib = ib_ref[0, 0]
    keep = (sc > t) | ((sc == t) & (iota <= ib))
    nm = (keep & (iota < N)).astype(F32)
    nm_ref[...] = nm
    h1p = h1_ref[...] * sc * nm
    h1p_ref[...] = h1p
    uv2_ref[...] = jnp.dot(h1p, w_ref[...], preferred_element_type=F32)
    nmtab_ref[...] = jnp.broadcast_to(nm, (NP, 16))
    a2_ref[...] = jnp.maximum(e_ref[...] * w0_ref[0, 0] + b0_ref[0, 0], 0.0)


_tc_d = pl.pallas_call(
    _tcd_body,
    out_shape=(jax.ShapeDtypeStruct((NP, 8), F32),
               jax.ShapeDtypeStruct((NP, 16), F32),
               jax.ShapeDtypeStruct((NP, 16), F32),
               jax.ShapeDtypeStruct((ER, 128), F32),
               jax.ShapeDtypeStruct((NP, 1), F32)),
)


def _tce_body(accm_ref, acco_ref, acci_ref, bias_ref, nm_ref, pw_ref,
              h2_ref, qtab_ref, rdin_ref):
    nm = nm_ref[...]
    m = accm_ref[:NP] + accm_ref[NP:]
    h2 = (m[:, :8] + m[:, 8:] + bias_ref[...]) * nm
    h2_ref[...] = h2
    dego = jnp.maximum((acco_ref[:NP, 0:1] + acco_ref[NP:, 0:1]) * nm, 1.0)
    degi = jnp.maximum((acci_ref[:NP, 0:1] + acci_ref[NP:, 0:1]) * nm, 1.0)
    rdin_ref[...] = jnp.broadcast_to(lax.rsqrt(degi), (NP, 16))
    q = jnp.sum(h2 * pw_ref[...], axis=1, keepdims=True)
    qtab_ref[...] = jnp.broadcast_to(q * lax.rsqrt(dego), (NP, 16))


_tc_e = pl.pallas_call(
    _tce_body,
    out_shape=(jax.ShapeDtypeStruct((NP, 8), F32),
               jax.ShapeDtypeStruct((NP, 16), F32),
               jax.ShapeDtypeStruct((NP, 16), F32)),
    compiler_params=pltpu.CompilerParams(vmem_limit_bytes=64 * 1024 * 1024),
)


def _tcf_body(acca_ref, rdin_ref, b_ref, nm_ref, score_ref, masked_ref):
    agg = acca_ref[:NP] + acca_ref[NP:]
    sv = agg * rdin_ref[...] + b_ref[0, 0]
    sc = 1.0 / (1.0 + jnp.exp(-sv))
    score_ref[...] = sc
    masked_ref[...] = jnp.where(nm_ref[...] > 0, sc, -1.0)


_tc_f = pl.pallas_call(
    _tcf_body,
    out_shape=(jax.ShapeDtypeStruct((NP, 16), F32),
               jax.ShapeDtypeStruct((NP, 16), F32)),
)


def _tcg_body(h1p_ref, nm1_ref, h2_ref, score2_ref, masked2_ref, t_ref,
              ib_ref, w1_ref, b1_ref, w2_ref, b2_ref, w3_ref, b3_ref,
              out_ref):
    h1p = h1p_ref[...]
    g1a = jnp.sum(h1p, axis=0, keepdims=True) * (1.0 / 5000.0)
    g1m = jnp.max(jnp.where(nm1_ref[...] > 0, h1p, -jnp.inf), axis=0,
                  keepdims=True)
    m2 = masked2_ref[:, 0:1]
    iota = lax.broadcasted_iota(jnp.int32, (NP, 1), 0)
    t = t_ref[0, 0]
    ib = ib_ref[0, 0]
    keep = (m2 > t) | ((m2 == t) & (iota <= ib))
    nm2 = (keep & (iota < N)).astype(F32)
    h2p = h2_ref[...] * score2_ref[:, 0:1] * nm2
    g2a = jnp.sum(h2p, axis=0, keepdims=True) * (1.0 / 2500.0)
    g2m = jnp.max(jnp.where(nm2 > 0, h2p, -jnp.inf), axis=0, keepdims=True)
    x = jnp.concatenate([g1a, g1m, g2a, g2m], axis=1)
    x = jnp.maximum(jnp.dot(x, w1_ref[...], preferred_element_type=F32)
                    + b1_ref[...], 0.0)
    x = jnp.maximum(jnp.dot(x, w2_ref[...], preferred_element_type=F32)
                    + b2_ref[...], 0.0)
    z = jnp.dot(x, w3_ref[...], preferred_element_type=F32) + b3_ref[...]
    zm = z - jnp.max(z, axis=1, keepdims=True)
    out_ref[...] = zm - jnp.log(jnp.sum(jnp.exp(zm), axis=1, keepdims=True))


_tc_g = pl.pallas_call(
    _tcg_body,
    out_shape=jax.ShapeDtypeStruct((1, 10), F32),
)


@jax.jit
def kernel(h, e, edge_index, n1_w0, n1_b0, n1_w1, n1_b1, conv1_bias, p1_w,
           p1_b, n2_w0, n2_b0, n2_w1, n2_b1, conv2_bias, p2_w, p2_b,
           fc1_w, fc1_b, fc2_w, fc2_b, fc3_w, fc3_b):
    pad_e = EP - E
    src2d = jnp.concatenate(
        [edge_index[0], jnp.full((pad_e,), N, jnp.int32)]).reshape(ER, 128)
    dst2d = jnp.concatenate(
        [edge_index[1], jnp.full((pad_e,), N, jnp.int32)]).reshape(ER, 128)
    hpad = jnp.pad(h, ((0, NP - N), (0, 0)))
    e2d = jnp.pad(e[:, 0], (0, pad_e)).reshape(ER, 128)
    ztab = jnp.zeros((NP, 16), F32)
    onestab = jnp.pad(jnp.ones((N, 16), F32), ((0, NP - N), (0, 0)))
    wcat1 = jnp.concatenate(
        [n1_w1.reshape(128, 8), n1_b1.reshape(128, 8)], axis=1)
    wcat2 = jnp.concatenate(
        [n2_w1.reshape(8, 8), n2_b1.reshape(8, 8)], axis=1)

    uv1, a1 = _tc_a(hpad, wcat1, e2d, n1_w0, n1_b0.reshape(1, 1))
    accm, acco, acci = _sc_conv(uv1, onestab, a1, src2d, dst2d, ztab)
    h1, qtab1, rdin1 = _tc_b(accm, acco, acci, conv1_bias.reshape(1, 8),
                             p1_w.reshape(1, 8))
    acca1 = _sc_agg(qtab1, src2d, dst2d, ztab)
    score1 = _tc_c(acca1, rdin1, p1_b.reshape(1, 1))
    vals1, idx1 = lax.top_k(score1[:N, 0], 5000)
    t1 = vals1[-1].reshape(1, 1)
    ib1 = idx1[-1].astype(jnp.int32).reshape(1, 1)
    h1p, uv2, nmtab1, a2, nm1 = _tc_d(h1, score1, t1, ib1, wcat2, e2d,
                                      n2_w0, n2_b0.reshape(1, 1))
    accm2, acco2, acci2 = _sc_conv(uv2, nmtab1, a2, src2d, dst2d, ztab)
    h2, qtab2, rdin2 = _tc_e(accm2, acco2, acci2, conv2_bias.reshape(1, 8),
                             nm1, p2_w.reshape(1, 8))
    acca2 = _sc_agg(qtab2, src2d, dst2d, ztab)
    score2, masked2 = _tc_f(acca2, rdin2, p2_b.reshape(1, 1), nm1)
    vals2, idx2 = lax.top_k(masked2[:N, 0], 2500)
    t2 = vals2[-1].reshape(1, 1)
    ib2 = idx2[-1].astype(jnp.int32).reshape(1, 1)
    x = _tc_g(h1p, nm1, h2, score2, masked2, t2, ib2,
              fc1_w, fc1_b.reshape(1, 64), fc2_w, fc2_b.reshape(1, 8),
              fc3_w, fc3_b.reshape(1, 10))
    return (x, vals1, vals2)


# speedup vs baseline: 11.1049x; 1.3196x over previous
import dataclasses
import functools
import jax
import jax.numpy as jnp
from jax import lax
from jax.experimental import pallas as pl
from jax.experimental.pallas import tpu as pltpu
from jax.experimental.pallas import tpu_sc as plsc

N = 10000
NP = 10240
E = 160000
EP = 163840
ER = EP // 128
WPW = ER // 32
ZR = NP // 16
F32 = jnp.float32

_mesh = plsc.VectorSubcoreMesh(core_axis_name="c", subcore_axis_name="s")

_sc_params = pltpu.CompilerParams()
for _f, _v in (("needs_layout_passes", False),
               ("use_tc_tiling_on_sc", False)):
    if _f in pltpu.CompilerParams.__dataclass_fields__:
        _sc_params = dataclasses.replace(_sc_params, **{_f: _v})


def _acc_sds():
    return jax.ShapeDtypeStruct((2 * NP, 16), F32)


@functools.partial(
    pl.kernel,
    out_type=(_acc_sds(), _acc_sds(), _acc_sds()),
    mesh=_mesh,
    compiler_params=_sc_params,
    scratch_types=[
        pltpu.VMEM_SHARED((NP, 16), F32),
        pltpu.VMEM_SHARED((NP, 16), F32),
        pltpu.VMEM_SHARED((NP, 16), F32),
        pltpu.VMEM((WPW, 128), jnp.int32),
        pltpu.VMEM((WPW, 128), jnp.int32),
        pltpu.VMEM((WPW, 128), F32),
        pltpu.VMEM((2, 128, 16), F32),
        pltpu.VMEM((2, 128, 16), F32),
        pltpu.VMEM((2, 128, 16), F32),
        pltpu.SemaphoreType.DMA((2,)),
        pltpu.SemaphoreType.DMA((2,)),
        pltpu.SemaphoreType.DMA((2,)),
    ],
)
def _sc_conv(uv_hbm, ind_hbm, a_hbm, src_hbm, dst_hbm, z_hbm,
             accm_hbm, acco_hbm, acci_hbm,
             shm, sho, shi, idx_s, idx_d, ab, rm2, ra2, rb2, sm, sa, sb):
    c = lax.axis_index("c")
    s = lax.axis_index("s")
    wid = s * 2 + c
    pltpu.sync_copy(z_hbm.at[pl.ds(s * ZR, ZR)], shm.at[pl.ds(s * ZR, ZR)])
    pltpu.sync_copy(z_hbm.at[pl.ds(s * ZR, ZR)], sho.at[pl.ds(s * ZR, ZR)])
    pltpu.sync_copy(z_hbm.at[pl.ds(s * ZR, ZR)], shi.at[pl.ds(s * ZR, ZR)])
    plsc.subcore_barrier()

    base = wid * WPW
    pltpu.sync_copy(src_hbm.at[pl.ds(base, WPW)], idx_s)
    pltpu.sync_copy(dst_hbm.at[pl.ds(base, WPW)], idx_d)
    pltpu.sync_copy(a_hbm.at[pl.ds(base, WPW)], ab)

    i16 = lax.broadcasted_iota(jnp.int32, (16,), 0)
    maskf = (i16 < 8).astype(F32)
    invf = 1.0 - maskf

    def start(jj, p):
        pltpu.async_copy(uv_hbm.at[idx_s.at[jj]], rm2.at[p], sm.at[p])
        pltpu.async_copy(ind_hbm.at[idx_s.at[jj]], ra2.at[p], sa.at[p])
        pltpu.async_copy(ind_hbm.at[idx_d.at[jj]], rb2.at[p], sb.at[p])

    def finish(jj, p):
        rm = rm2.at[p]
        ra = ra2.at[p]
        rb = rb2.at[p]
        pltpu.make_async_copy(uv_hbm.at[idx_s.at[jj]], rm, sm.at[p]).wait()
        pltpu.make_async_copy(ind_hbm.at[idx_s.at[jj]], ra, sa.at[p]).wait()
        pltpu.make_async_copy(ind_hbm.at[idx_d.at[jj]], rb, sb.at[p]).wait()

        @pl.loop(0, 8)
        def _(g):
            av = ab[jj, pl.ds(g * 16, 16)]
            for k in range(16):
                i = g * 16 + k
                rm[i, :] = rm[i, :] * (maskf * av[k] + invf)

        pltpu.sync_copy(rm, shm.at[idx_d.at[jj]], add=True)
        pltpu.sync_copy(ra, shi.at[idx_d.at[jj]], add=True)
        pltpu.sync_copy(rb, sho.at[idx_s.at[jj]], add=True)

    start(0, 0)

    @pl.loop(0, WPW // 2)
    def _(jp):
        j0 = 2 * jp

        @pl.when(j0 + 1 < WPW)
        def _():
            start(j0 + 1, 1)

        finish(j0, 0)

        @pl.when(j0 + 2 < WPW)
        def _():
            start(j0 + 2, 0)

        finish(j0 + 1, 1)

    plsc.subcore_barrier()
    off = c * NP + s * ZR
    pltpu.sync_copy(shm.at[pl.ds(s * ZR, ZR)], accm_hbm.at[pl.ds(off, ZR)])
    pltpu.sync_copy(sho.at[pl.ds(s * ZR, ZR)], acco_hbm.at[pl.ds(off, ZR)])
    pltpu.sync_copy(shi.at[pl.ds(s * ZR, ZR)], acci_hbm.at[pl.ds(off, ZR)])


@functools.partial(
    pl.kernel,
    out_type=_acc_sds(),
    mesh=_mesh,
    compiler_params=_sc_params,
    scratch_types=[
        pltpu.VMEM_SHARED((NP, 16), F32),
        pltpu.VMEM((WPW, 128), jnp.int32),
        pltpu.VMEM((WPW, 128), jnp.int32),
        pltpu.VMEM((2, 128, 16), F32),
        pltpu.SemaphoreType.DMA((2,)),
    ],
)
def _sc_agg(q_hbm, src_hbm, dst_hbm, z_hbm, acca_hbm, sha, idx_s, idx_d,
            rq2, sq):
    c = lax.axis_index("c")
    s = lax.axis_index("s")
    wid = s * 2 + c
    pltpu.sync_copy(z_hbm.at[pl.ds(s * ZR, ZR)], sha.at[pl.ds(s * ZR, ZR)])
    plsc.subcore_barrier()

    base = wid * WPW
    pltpu.sync_copy(src_hbm.at[pl.ds(base, WPW)], idx_s)
    pltpu.sync_copy(dst_hbm.at[pl.ds(base, WPW)], idx_d)

    def start(jj, p):
        pltpu.async_copy(q_hbm.at[idx_s.at[jj]], rq2.at[p], sq.at[p])

    def finish(jj, p):
        pltpu.make_async_copy(q_hbm.at[idx_s.at[jj]], rq2.at[p],
                              sq.at[p]).wait()
        pltpu.sync_copy(rq2.at[p], sha.at[idx_d.at[jj]], add=True)

    start(0, 0)

    @pl.loop(0, WPW // 2)
    def _(jp):
        j0 = 2 * jp

        @pl.when(j0 + 1 < WPW)
        def _():
            start(j0 + 1, 1)

        finish(j0, 0)

        @pl.when(j0 + 2 < WPW)
        def _():
            start(j0 + 2, 0)

        finish(j0 + 1, 1)

    plsc.subcore_barrier()
    off = c * NP + s * ZR
    pltpu.sync_copy(sha.at[pl.ds(s * ZR, ZR)], acca_hbm.at[pl.ds(off, ZR)])


def _tca_body(h_ref, w_ref, e_ref, w0_ref, b0_ref, uv_ref, a_ref):
    uv_ref[...] = jnp.dot(h_ref[...], w_ref[...], preferred_element_type=F32)
    a_ref[...] = jnp.maximum(e_ref[...] * w0_ref[0, 0] + b0_ref[0, 0], 0.0)


_tc_a = pl.pallas_call(
    _tca_body,
    out_shape=(jax.ShapeDtypeStruct((NP, 16), F32),
               jax.ShapeDtypeStruct((ER, 128), F32)),
)


def _tcb_body(accm_ref, acco_ref, acci_ref, bias_ref, pw_ref,
              h1_ref, qtab_ref, rdin_ref):
    m = accm_ref[:NP] + accm_ref[NP:]
    h1 = m[:, :8] + m[:, 8:] + bias_ref[...]
    h1_ref[...] = h1
    dego = jnp.maximum(acco_ref[:NP, 0:1] + acco_ref[NP:, 0:1], 1.0)
    degi = jnp.maximum(acci_ref[:NP, 0:1] + acci_ref[NP:, 0:1], 1.0)
    rdin_ref[...] = jnp.broadcast_to(lax.rsqrt(degi), (NP, 16))
    q = jnp.sum(h1 * pw_ref[...], axis=1, keepdims=True)
    qtab_ref[...] = jnp.broadcast_to(q * lax.rsqrt(dego), (NP, 16))


_tc_b = pl.pallas_call(
    _tcb_body,
    out_shape=(jax.ShapeDtypeStruct((NP, 8), F32),
               jax.ShapeDtypeStruct((NP, 16), F32),
               jax.ShapeDtypeStruct((NP, 16), F32)),
)


def _tcc_body(acca_ref, rdin_ref, b_ref, score_ref):
    agg = acca_ref[:NP] + acca_ref[NP:]
    sv = agg * rdin_ref[...] + b_ref[0, 0]
    score_ref[...] = 1.0 / (1.0 + jnp.exp(-sv))


_tc_c = pl.pallas_call(
    _tcc_body,
    out_shape=jax.ShapeDtypeStruct((NP, 16), F32),
)


def _tcd_body(h1_ref, score_ref, t_ref, ib_ref, w_ref, e_ref, w0_ref, b0_ref,
              h1p_ref, uv2_ref, nmtab_ref, a2_ref, nm_ref):
    sc = score_ref[:, 0:1]
    iota = lax.broadcasted_iota(jnp.int32, (NP, 1), 0)
    t = t_ref[0, 0]
    ib = ib_ref[0, 0]
    keep = (sc > t) | ((sc == t) & (iota <= ib))
    nm = (keep & (iota < N)).astype(F32)
    nm_ref[...] = nm
    h1p = h1_ref[...] * sc * nm
    h1p_ref[...] = h1p
    uv2_ref[...] = jnp.dot(h1p, w_ref[...], preferred_element_type=F32)
    nmtab_ref[...] = jnp.broadcast_to(nm, (NP, 16))
    a2_ref[...] = jnp.maximum(e_ref[...] * w0_ref[0, 0] + b0_ref[0, 0], 0.0)


_tc_d = pl.pallas_call(
    _tcd_body,
    out_shape=(jax.ShapeDtypeStruct((NP, 8), F32),
               jax.ShapeDtypeStruct((NP, 16), F32),
               jax.ShapeDtypeStruct((NP, 16), F32),
               jax.ShapeDtypeStruct((ER, 128), F32),
               jax.ShapeDtypeStruct((NP, 1), F32)),
)


def _tce_body(accm_ref, acco_ref, acci_ref, bias_ref, nm_ref, pw_ref,
              h2_ref, qtab_ref, rdin_ref):
    nm = nm_ref[...]
    m = accm_ref[:NP] + accm_ref[NP:]
    h2 = (m[:, :8] + m[:, 8:] + bias_ref[...]) * nm
    h2_ref[...] = h2
    dego = jnp.maximum((acco_ref[:NP, 0:1] + acco_ref[NP:, 0:1]) * nm, 1.0)
    degi = jnp.maximum((acci_ref[:NP, 0:1] + acci_ref[NP:, 0:1]) * nm, 1.0)
    rdin_ref[...] = jnp.broadcast_to(lax.rsqrt(degi), (NP, 16))
    q = jnp.sum(h2 * pw_ref[...], axis=1, keepdims=True)
    qtab_ref[...] = jnp.broadcast_to(q * lax.rsqrt(dego), (NP, 16))


_tc_e = pl.pallas_call(
    _tce_body,
    out_shape=(jax.ShapeDtypeStruct((NP, 8), F32),
               jax.ShapeDtypeStruct((NP, 16), F32),
               jax.ShapeDtypeStruct((NP, 16), F32)),
    compiler_params=pltpu.CompilerParams(vmem_limit_bytes=64 * 1024 * 1024),
)


def _tcf_body(acca_ref, rdin_ref, b_ref, nm_ref, score_ref, masked_ref):
    agg = acca_ref[:NP] + acca_ref[NP:]
    sv = agg * rdin_ref[...] + b_ref[0, 0]
    sc = 1.0 / (1.0 + jnp.exp(-sv))
    score_ref[...] = sc
    masked_ref[...] = jnp.where(nm_ref[...] > 0, sc, -1.0)


_tc_f = pl.pallas_call(
    _tcf_body,
    out_shape=(jax.ShapeDtypeStruct((NP, 16), F32),
               jax.ShapeDtypeStruct((NP, 16), F32)),
)


def _tcg_body(h1p_ref, nm1_ref, h2_ref, score2_ref, masked2_ref, t_ref,
              ib_ref, w1_ref, b1_ref, w2_ref, b2_ref, w3_ref, b3_ref,
              out_ref):
    h1p = h1p_ref[...]
    g1a = jnp.sum(h1p, axis=0, keepdims=True) * (1.0 / 5000.0)
    g1m = jnp.max(jnp.where(nm1_ref[...] > 0, h1p, -jnp.inf), axis=0,
                  keepdims=True)
    m2 = masked2_ref[:, 0:1]
    iota = lax.broadcasted_iota(jnp.int32, (NP, 1), 0)
    t = t_ref[0, 0]
    ib = ib_ref[0, 0]
    keep = (m2 > t) | ((m2 == t) & (iota <= ib))
    nm2 = (keep & (iota < N)).astype(F32)
    h2p = h2_ref[...] * score2_ref[:, 0:1] * nm2
    g2a = jnp.sum(h2p, axis=0, keepdims=True) * (1.0 / 2500.0)
    g2m = jnp.max(jnp.where(nm2 > 0, h2p, -jnp.inf), axis=0, keepdims=True)
    x = jnp.concatenate([g1a, g1m, g2a, g2m], axis=1)
    x = jnp.maximum(jnp.dot(x, w1_ref[...], preferred_element_type=F32)
                    + b1_ref[...], 0.0)
    x = jnp.maximum(jnp.dot(x, w2_ref[...], preferred_element_type=F32)
                    + b2_ref[...], 0.0)
    z = jnp.dot(x, w3_ref[...], preferred_element_type=F32) + b3_ref[...]
    zm = z - jnp.max(z, axis=1, keepdims=True)
    out_ref[...] = zm - jnp.log(jnp.sum(jnp.exp(zm), axis=1, keepdims=True))


_tc_g = pl.pallas_call(
    _tcg_body,
    out_shape=jax.ShapeDtypeStruct((1, 10), F32),
)


@jax.jit
def kernel(h, e, edge_index, n1_w0, n1_b0, n1_w1, n1_b1, conv1_bias, p1_w,
           p1_b, n2_w0, n2_b0, n2_w1, n2_b1, conv2_bias, p2_w, p2_b,
           fc1_w, fc1_b, fc2_w, fc2_b, fc3_w, fc3_b):
    pad_e = EP - E
    src2d = jnp.concatenate(
        [edge_index[0], jnp.full((pad_e,), N, jnp.int32)]).reshape(ER, 128)
    dst2d = jnp.concatenate(
        [edge_index[1], jnp.full((pad_e,), N, jnp.int32)]).reshape(ER, 128)
    hpad = jnp.pad(h, ((0, NP - N), (0, 0)))
    e2d = jnp.pad(e[:, 0], (0, pad_e)).reshape(ER, 128)
    ztab = jnp.zeros((NP, 16), F32)
    onestab = jnp.pad(jnp.ones((N, 16), F32), ((0, NP - N), (0, 0)))
    wcat1 = jnp.concatenate(
        [n1_w1.reshape(128, 8), n1_b1.reshape(128, 8)], axis=1)
    wcat2 = jnp.concatenate(
        [n2_w1.reshape(8, 8), n2_b1.reshape(8, 8)], axis=1)

    uv1, a1 = _tc_a(hpad, wcat1, e2d, n1_w0, n1_b0.reshape(1, 1))
    accm, acco, acci = _sc_conv(uv1, onestab, a1, src2d, dst2d, ztab)
    h1, qtab1, rdin1 = _tc_b(accm, acco, acci, conv1_bias.reshape(1, 8),
                             p1_w.reshape(1, 8))
    acca1 = _sc_agg(qtab1, src2d, dst2d, ztab)
    score1 = _tc_c(acca1, rdin1, p1_b.reshape(1, 1))
    vals1, idx1 = lax.top_k(score1[:N, 0], 5000)
    t1 = vals1[-1].reshape(1, 1)
    ib1 = idx1[-1].astype(jnp.int32).reshape(1, 1)
    h1p, uv2, nmtab1, a2, nm1 = _tc_d(h1, score1, t1, ib1, wcat2, e2d,
                                      n2_w0, n2_b0.reshape(1, 1))
    accm2, acco2, acci2 = _sc_conv(uv2, nmtab1, a2, src2d, dst2d, ztab)
    h2, qtab2, rdin2 = _tc_e(accm2, acco2, acci2, conv2_bias.reshape(1, 8),
                             nm1, p2_w.reshape(1, 8))
    acca2 = _sc_agg(qtab2, src2d, dst2d, ztab)
    score2, masked2 = _tc_f(acca2, rdin2, p2_b.reshape(1, 1), nm1)
    vals2, idx2 = lax.top_k(masked2[:N, 0], 2500)
    t2 = vals2[-1].reshape(1, 1)
    ib2 = idx2[-1].astype(jnp.int32).reshape(1, 1)
    x = _tc_g(h1p, nm1, h2, score2, masked2, t2, ib2,
              fc1_w, fc1_b.reshape(1, 64), fc2_w, fc2_b.reshape(1, 8),
              fc3_w, fc3_b.reshape(1, 10))
    return (x, vals1, vals2)
